# baseline (speedup 1.0000x reference)
import functools

import jax
import jax.numpy as jnp
from jax import lax
from jax.experimental import pallas as pl
from jax.experimental.pallas import tpu as pltpu
from jax.experimental.pallas import tpu_sc as plsc

B = 4096
NF = 26
V = 100000
D = 64

NC = 2
NS = 16
NW = NC * NS
ROWS = B * NF
RPW = ROWS // NW
CHUNK = 128
NCH = RPW // CHUNK

BB = 512
NB = B // BB

CH2 = 128
NCH3 = RPW // CH2


@functools.cache
def _make_sc_gather():
    mesh = plsc.VectorSubcoreMesh(core_axis_name="c", subcore_axis_name="s")

    @functools.partial(
        pl.kernel,
        mesh=mesh,
        out_type=jax.ShapeDtypeStruct((NF // 2, B, 2 * D), jnp.float32),
        scratch_types=[
            pltpu.VMEM((NF, 128), jnp.int32),
            pltpu.VMEM((2, CH2), jnp.int32),
            pltpu.VMEM((2, CH2, 2 * D), jnp.float32),
            pltpu.VMEM((CH2 // 2, 2 * D), jnp.float32),
            pltpu.SemaphoreType.DMA,
            pltpu.SemaphoreType.DMA,
            pltpu.SemaphoreType.DMA,
        ],
    )
    def _sc_gather(tab_hbm, idx_hbm, out_hbm, idx_all, idx_g, stage_v,
                   outb_v, sem_i, sem_g0, sem_g1):
        wid = lax.axis_index("s") * NC + lax.axis_index("c")
        gsems = (sem_g0, sem_g1)

        for k in range(NF):
            pltpu.async_copy(idx_hbm.at[k, pl.ds(wid * 128, 128)],
                             idx_all.at[k], sem_i)
        for k in range(NF):
            pltpu.make_async_copy(idx_hbm.at[0, pl.ds(0, 128)],
                                  idx_all.at[0], sem_i).wait()

        def build_issue(c, slot):
            jrow = c // 2
            cc = c % 2
            for g in range(CH2 // 16):
                kk = 2 * jrow + g // 4
                col = cc * 64 + (g % 4) * 16
                v16 = idx_all[kk, pl.ds(col, 16)]
                idx_g[slot, pl.ds(g * 16, 16)] = (v16 >> 1) + kk * (V // 2)
            return pltpu.async_copy(tab_hbm.at[idx_g.at[slot]],
                                    stage_v.at[slot], gsems[slot])

        def select_write(c, slot):
            jrow = c // 2
            cc = c % 2

            def sel(g, _):
                kk = 2 * jrow + g // 4
                k2 = g // 4
                col = cc * 64 + (g % 4) * 16
                v16 = idx_all[kk, pl.ds(col, 16)]
                for j in range(16):
                    r = g * 16 + j
                    rr = (g % 4) * 16 + j
                    s = v16[j] & 1
                    for sub in range(4):
                        outb_v[rr, pl.ds(k2 * D + sub * 16, 16)] = (
                            stage_v[slot, r, pl.ds(s * D + sub * 16, 16)])
                return 0

            lax.fori_loop(0, CH2 // 16, sel, 0)
            pltpu.sync_copy(
                outb_v,
                out_hbm.at[jrow, pl.ds(wid * 128 + cc * (CH2 // 2),
                                       CH2 // 2)])

        cps = [None, None]
        cps[0] = build_issue(0, 0)
        for c in range(NCH3):
            slot = c % 2
            if c + 1 < NCH3:
                cps[1 - slot] = build_issue(c + 1, 1 - slot)
            cps[slot].wait()
            select_write(c, slot)

    return _sc_gather


def _tc_dense_body(dxT_r, ly_r, bw0_r, bb0_r, bw1_r, bb1_r, bw2_r, bb2_r,
                   w0x_r, w0z_r, tb0_r, tw1_r, tb1_r, tw2_r, tb2_r,
                   out_r, zscr):
    f32 = jnp.float32
    mm = functools.partial(jnp.dot, preferred_element_type=f32)
    x0 = jnp.maximum(mm(bw0_r[...], dxT_r[...]) + bb0_r[...][:, None], 0.0)
    x1 = jnp.maximum(mm(bw1_r[...], x0) + bb1_r[...][:, None], 0.0)
    xT = jnp.maximum(mm(bw2_r[...], x1) + bb2_r[...][:, None], 0.0)
    ii = lax.broadcasted_iota(jnp.int32, (BB, BB), 0)
    jj = lax.broadcasted_iota(jnp.int32, (BB, BB), 1)
    ident = jnp.where(ii == jj, 1.0, 0.0).astype(f32)
    lyJ = ly_r[...]
    lyT = jnp.concatenate(
        [lax.dot_general(lyJ[j], ident, (((0,), (0,)), ((), ())),
                         preferred_element_type=f32)
         for j in range(NF // 2)], axis=0)
    ly3 = lyT.reshape(NF, D, BB)
    qx = jnp.sum(ly3 * xT[None], axis=1)
    for i in range(1, NF + 1):
        s = i * (i - 1) // 2
        zscr[s:s + 1, :] = qx[i - 1:i, :]
        a = i - 1
        if a >= 1:
            qa = jnp.sum(ly3[:a] * ly3[a][None], axis=1)
            zscr[s + 1:s + 1 + a, :] = qa
    zscr[351:352, :] = jnp.zeros((1, BB), f32)
    Z = zscr[...]
    r1 = mm(w0x_r[...], xT) + mm(w0z_r[...], Z) + tb0_r[...][:, None]
    z1 = jnp.maximum(r1, 0.0)
    z2 = jnp.maximum(mm(tw1_r[...], z1) + tb1_r[...][:, None], 0.0)
    z3 = mm(tw2_r[...], z2) + tb2_r[...][:, None]
    pid = pl.program_id(0)
    out_r[pl.ds(pid, 1), :] = jax.nn.sigmoid(z3)


def _tc_dense(dxT, ly2, bot_W0, bot_b0, bot_W1, bot_b1, bot_W2, bot_b2,
              w0x, w0z, top_b0, top_W1, top_b1, top_W2, top_b2):
    def full(shape):
        return pl.BlockSpec(shape, lambda *_: (0,) * len(shape))
    return pl.pallas_call(
        _tc_dense_body,
        grid=(NB,),
        in_specs=[
            pl.BlockSpec((13, BB), lambda i: (0, i)),
            pl.BlockSpec((NF // 2, BB, 2 * D), lambda i: (0, i, 0)),
            full((512, 13)), full((512,)),
            full((256, 512)), full((256,)),
            full((64, 256)), full((64,)),
            full((512, 64)), full((512, 352)), full((512,)),
            full((256, 512)), full((256,)),
            full((1, 256)), full((1,)),
        ],
        out_specs=pl.BlockSpec((NB, BB), lambda i: (0, 0)),
        out_shape=jax.ShapeDtypeStruct((NB, BB), jnp.float32),
        scratch_shapes=[pltpu.VMEM((352, BB), jnp.float32)],
    )(dxT, ly2, bot_W0, bot_b0, bot_W1, bot_b1, bot_W2, bot_b2,
      w0x, w0z, top_b0, top_W1, top_b1, top_W2, top_b2)


def kernel(dense_x, lS_o, lS_i, emb, bot_W0, bot_b0, bot_W1, bot_b1,
           bot_W2, bot_b2, top_W0, top_b0, top_W1, top_b1, top_W2, top_b2):
    del lS_o
    tab = emb.reshape(NF * V // 2, 2 * D)
    ly2 = _make_sc_gather()(tab, lS_i)
    dxT = dense_x.T
    w0x = top_W0[:, :D]
    w0z = jnp.pad(top_W0[:, D:], ((0, 0), (0, 1)))
    out = _tc_dense(dxT, ly2, bot_W0, bot_b0, bot_W1, bot_b1, bot_W2, bot_b2,
                    w0x, w0z, top_b0, top_W1, top_b1, top_W2, top_b2)
    return out.reshape(B, 1)


# speedup vs baseline: 2.0599x; 2.0599x over previous
import functools

import jax
import jax.numpy as jnp
from jax import lax
from jax.experimental import pallas as pl
from jax.experimental.pallas import tpu as pltpu
from jax.experimental.pallas import tpu_sc as plsc

B = 4096
NF = 26
V = 100000
D = 64

NC = 2
NS = 16
NW = NC * NS
ROWS = B * NF
RPW = ROWS // NW
CHUNK = 128
NCH = RPW // CHUNK

BB = 512
NB = B // BB

CHS = 64
NCH2 = RPW // CHS


@functools.cache
def _make_sc_gather():
    mesh = plsc.VectorSubcoreMesh(core_axis_name="c", subcore_axis_name="s")

    @functools.partial(
        pl.kernel,
        mesh=mesh,
        out_type=jax.ShapeDtypeStruct((NF // 2, B, 2 * D), jnp.float32),
        scratch_types=[
            pltpu.VMEM((NF, 128), jnp.int32),
            pltpu.VMEM((CHS, 8, D), jnp.float32),
            pltpu.VMEM((CHS // 2, 2 * D), jnp.float32),
            pltpu.SemaphoreType.DMA,
            pltpu.SemaphoreType.DMA,
        ],
    )
    def _sc_gather(emb_hbm, idx_hbm, out_hbm, idx_all, stage_v, outb_v,
                   sem_g, sem_i):
        wid = lax.axis_index("s") * NC + lax.axis_index("c")

        for k in range(NF):
            pltpu.async_copy(idx_hbm.at[k, pl.ds(wid * 128, 128)],
                             idx_all.at[k], sem_i)
        for k in range(NF):
            pltpu.make_async_copy(idx_hbm.at[0, pl.ds(0, 128)],
                                  idx_all.at[0], sem_i).wait()

        def chunk_body(c, _):
            jrow = c // 4
            cc = c % 4

            def issue(g, _):
                kk = 2 * jrow + g // 2
                p16 = idx_all[kk, pl.ds(cc * 32 + (g % 2) * 16, 16)]
                for j in range(16):
                    v = p16[j]
                    q = v >> 3
                    pltpu.async_copy(emb_hbm.at[kk, q],
                                     stage_v.at[g * 16 + j], sem_g)
                return 0

            def drain(r, _):
                pltpu.make_async_copy(emb_hbm.at[0, 0],
                                      stage_v.at[0], sem_g).wait()
                return 0

            def select(g, _):
                kk = 2 * jrow + g // 2
                k2 = g // 2
                p16 = idx_all[kk, pl.ds(cc * 32 + (g % 2) * 16, 16)]
                for j in range(16):
                    r = g * 16 + j
                    rr = (g % 2) * 16 + j
                    s = p16[j] & 7
                    for sub in range(4):
                        outb_v[rr, pl.ds(k2 * D + sub * 16, 16)] = (
                            stage_v[r, s, pl.ds(sub * 16, 16)])
                return 0

            lax.fori_loop(0, CHS // 16, issue, 0)
            lax.fori_loop(0, CHS, drain, 0, unroll=8)
            lax.fori_loop(0, CHS // 16, select, 0)
            pltpu.sync_copy(
                outb_v,
                out_hbm.at[jrow, pl.ds(wid * 128 + cc * (CHS // 2),
                                       CHS // 2)])
            return 0

        lax.fori_loop(0, NCH2, chunk_body, 0)

    return _sc_gather


def _tc_dense_body(dxT_r, ly_r, bw0_r, bb0_r, bw1_r, bb1_r, bw2_r, bb2_r,
                   w0x_r, w0z_r, tb0_r, tw1_r, tb1_r, tw2_r, tb2_r,
                   out_r, zscr):
    f32 = jnp.float32
    mm = functools.partial(jnp.dot, preferred_element_type=f32)
    x0 = jnp.maximum(mm(bw0_r[...], dxT_r[...]) + bb0_r[...][:, None], 0.0)
    x1 = jnp.maximum(mm(bw1_r[...], x0) + bb1_r[...][:, None], 0.0)
    xT = jnp.maximum(mm(bw2_r[...], x1) + bb2_r[...][:, None], 0.0)
    ii = lax.broadcasted_iota(jnp.int32, (BB, BB), 0)
    jj = lax.broadcasted_iota(jnp.int32, (BB, BB), 1)
    ident = jnp.where(ii == jj, 1.0, 0.0).astype(f32)
    lyJ = ly_r[...]
    lyT = jnp.concatenate(
        [lax.dot_general(lyJ[j], ident, (((0,), (0,)), ((), ())),
                         preferred_element_type=f32)
         for j in range(NF // 2)], axis=0)
    ly3 = lyT.reshape(NF, D, BB)
    qx = jnp.sum(ly3 * xT[None], axis=1)
    for i in range(1, NF + 1):
        s = i * (i - 1) // 2
        zscr[s:s + 1, :] = qx[i - 1:i, :]
        a = i - 1
        if a >= 1:
            qa = jnp.sum(ly3[:a] * ly3[a][None], axis=1)
            zscr[s + 1:s + 1 + a, :] = qa
    zscr[351:352, :] = jnp.zeros((1, BB), f32)
    Z = zscr[...]
    r1 = mm(w0x_r[...], xT) + mm(w0z_r[...], Z) + tb0_r[...][:, None]
    z1 = jnp.maximum(r1, 0.0)
    z2 = jnp.maximum(mm(tw1_r[...], z1) + tb1_r[...][:, None], 0.0)
    z3 = mm(tw2_r[...], z2) + tb2_r[...][:, None]
    pid = pl.program_id(0)
    out_r[pl.ds(pid, 1), :] = jax.nn.sigmoid(z3)


def _tc_dense(dxT, ly2, bot_W0, bot_b0, bot_W1, bot_b1, bot_W2, bot_b2,
              w0x, w0z, top_b0, top_W1, top_b1, top_W2, top_b2):
    def full(shape):
        return pl.BlockSpec(shape, lambda *_: (0,) * len(shape))
    return pl.pallas_call(
        _tc_dense_body,
        grid=(NB,),
        in_specs=[
            pl.BlockSpec((13, BB), lambda i: (0, i)),
            pl.BlockSpec((NF // 2, BB, 2 * D), lambda i: (0, i, 0)),
            full((512, 13)), full((512,)),
            full((256, 512)), full((256,)),
            full((64, 256)), full((64,)),
            full((512, 64)), full((512, 352)), full((512,)),
            full((256, 512)), full((256,)),
            full((1, 256)), full((1,)),
        ],
        out_specs=pl.BlockSpec((NB, BB), lambda i: (0, 0)),
        out_shape=jax.ShapeDtypeStruct((NB, BB), jnp.float32),
        scratch_shapes=[pltpu.VMEM((352, BB), jnp.float32)],
    )(dxT, ly2, bot_W0, bot_b0, bot_W1, bot_b1, bot_W2, bot_b2,
      w0x, w0z, top_b0, top_W1, top_b1, top_W2, top_b2)


def kernel(dense_x, lS_o, lS_i, emb, bot_W0, bot_b0, bot_W1, bot_b1,
           bot_W2, bot_b2, top_W0, top_b0, top_W1, top_b1, top_W2, top_b2):
    del lS_o
    emb4 = emb.reshape(NF, V // 8, 8, D)
    ly2 = _make_sc_gather()(emb4, lS_i)
    dxT = dense_x.T
    w0x = top_W0[:, :D]
    w0z = jnp.pad(top_W0[:, D:], ((0, 0), (0, 1)))
    out = _tc_dense(dxT, ly2, bot_W0, bot_b0, bot_W1, bot_b1, bot_W2, bot_b2,
                    w0x, w0z, top_b0, top_W1, top_b1, top_W2, top_b2)
    return out.reshape(B, 1)
